# Initial kernel scaffold; baseline (speedup 1.0000x reference)
import jax
import jax.numpy as jnp
from jax.experimental import pallas as pl


def kernel(x, adj1, adj2, adj3, adj4, adj5, adj6, W1, b1, W2, b2, W3, b3, Wd, bd):
    raise NotImplementedError("write your pallas kernel here")


# speedup vs baseline: 1.0301x; 1.0301x over previous
import jax
import jax.numpy as jnp
from jax.experimental import pallas as pl
from jax.experimental.pallas import tpu as pltpu

BM1 = 200
BM2 = 400


def _xw_body(x_ref, w_ref, o_ref):
    o_ref[...] = jnp.dot(
        x_ref[...].astype(jnp.bfloat16), w_ref[...],
        preferred_element_type=jnp.float32).astype(jnp.bfloat16)


def _layer1_body(adj_ref, t_ref, b_ref, wn_ref, o_ref, adjb_ref):
    a = adj_ref[...].astype(jnp.bfloat16)
    adjb_ref[...] = a
    acc = jnp.dot(a, t_ref[...], preferred_element_type=jnp.float32)
    h = jnp.maximum(acc + b_ref[0, :], 0.0)
    o_ref[...] = jnp.dot(h.astype(jnp.bfloat16), wn_ref[...],
                         preferred_element_type=jnp.float32).astype(jnp.bfloat16)


def _layer_body(adj_ref, t_ref, b_ref, wn_ref, o_ref):
    acc = jnp.dot(adj_ref[...], t_ref[...], preferred_element_type=jnp.float32)
    h = jnp.maximum(acc + b_ref[0, :], 0.0)
    o_ref[...] = jnp.dot(h.astype(jnp.bfloat16), wn_ref[...],
                         preferred_element_type=jnp.float32).astype(jnp.bfloat16)


def _final_body(adj_ref, t_ref, b_ref, wn_ref, bd_ref, o_ref):
    acc = jnp.dot(adj_ref[...], t_ref[...], preferred_element_type=jnp.float32)
    h = jnp.maximum(acc + b_ref[0, :], 0.0)
    o_ref[...] = jnp.dot(h.astype(jnp.bfloat16), wn_ref[...],
                         preferred_element_type=jnp.float32) + bd_ref[0, :]


def _layer1(adj, t, b, w_next):
    n, f = t.shape
    fo = w_next.shape[1]
    return pl.pallas_call(
        _layer1_body,
        grid=(n // BM1,),
        in_specs=[
            pl.BlockSpec((BM1, n), lambda m: (m, 0)),
            pl.BlockSpec((n, f), lambda m: (0, 0)),
            pl.BlockSpec((1, f), lambda m: (0, 0)),
            pl.BlockSpec((f, fo), lambda m: (0, 0)),
        ],
        out_specs=[
            pl.BlockSpec((BM1, fo), lambda m: (m, 0)),
            pl.BlockSpec((BM1, n), lambda m: (m, 0)),
        ],
        out_shape=[
            jax.ShapeDtypeStruct((n, fo), jnp.bfloat16),
            jax.ShapeDtypeStruct((n, n), jnp.bfloat16),
        ],
        compiler_params=pltpu.CompilerParams(
            dimension_semantics=("arbitrary",)),
    )(adj, t, b, w_next)


def _layer(adj, t, b, w_next):
    n, f = t.shape
    fo = w_next.shape[1]
    return pl.pallas_call(
        _layer_body,
        grid=(n // BM2,),
        in_specs=[
            pl.BlockSpec((BM2, n), lambda m: (m, 0)),
            pl.BlockSpec((n, f), lambda m: (0, 0)),
            pl.BlockSpec((1, f), lambda m: (0, 0)),
            pl.BlockSpec((f, fo), lambda m: (0, 0)),
        ],
        out_specs=pl.BlockSpec((BM2, fo), lambda m: (m, 0)),
        out_shape=jax.ShapeDtypeStruct((n, fo), jnp.bfloat16),
        compiler_params=pltpu.CompilerParams(
            dimension_semantics=("arbitrary",)),
    )(adj, t, b, w_next)


def _final_layer(adj, t, b, wd, bd):
    n, f = t.shape
    fo = wd.shape[1]
    return pl.pallas_call(
        _final_body,
        grid=(n // BM2,),
        in_specs=[
            pl.BlockSpec((BM2, n), lambda m: (m, 0)),
            pl.BlockSpec((n, f), lambda m: (0, 0)),
            pl.BlockSpec((1, f), lambda m: (0, 0)),
            pl.BlockSpec((f, fo), lambda m: (0, 0)),
            pl.BlockSpec((1, fo), lambda m: (0, 0)),
        ],
        out_specs=pl.BlockSpec((BM2, fo), lambda m: (m, 0)),
        out_shape=jax.ShapeDtypeStruct((n, fo), jnp.float32),
        compiler_params=pltpu.CompilerParams(
            dimension_semantics=("arbitrary",)),
    )(adj, t, b, wd, bd)


def _xw(x, w):
    n, f = x.shape
    fo = w.shape[1]
    return pl.pallas_call(
        _xw_body,
        grid=(n // BM2,),
        in_specs=[
            pl.BlockSpec((BM2, f), lambda m: (m, 0)),
            pl.BlockSpec((f, fo), lambda m: (0, 0)),
        ],
        out_specs=pl.BlockSpec((BM2, fo), lambda m: (m, 0)),
        out_shape=jax.ShapeDtypeStruct((n, fo), jnp.bfloat16),
        compiler_params=pltpu.CompilerParams(
            dimension_semantics=("arbitrary",)),
    )(x, w)


def kernel(x, adj1, adj2, adj3, adj4, adj5, adj6, W1, b1, W2, b2, W3, b3,
           Wd, bd):
    w1b, w2b, w3b, wdb = (w.astype(jnp.bfloat16) for w in (W1, W2, W3, Wd))
    b1r = b1.reshape(1, -1)
    b2r = b2.reshape(1, -1)
    b3r = b3.reshape(1, -1)
    bdr = bd.reshape(1, -1)

    t1 = _xw(x, w1b)
    t2, adj_b = _layer1(adj5, t1, b1r, w2b)
    t3 = _layer(adj_b, t2, b2r, w3b)
    return _final_layer(adj_b, t3, b3r, wdb, bdr)


# speedup vs baseline: 1.1217x; 1.0889x over previous
import jax
import jax.numpy as jnp
from jax.experimental import pallas as pl
from jax.experimental.pallas import tpu as pltpu

BM1 = 400
BM2 = 800


def _xw_body(x_ref, w_ref, o_ref):
    o_ref[...] = jnp.dot(
        x_ref[...].astype(jnp.bfloat16), w_ref[...],
        preferred_element_type=jnp.float32).astype(jnp.bfloat16)


def _layer1_body(adj_ref, t_ref, b_ref, wn_ref, o_ref, adjb_ref):
    a = adj_ref[...].astype(jnp.bfloat16)
    adjb_ref[...] = a
    acc = jnp.dot(a, t_ref[...], preferred_element_type=jnp.float32)
    h = jnp.maximum(acc + b_ref[0, :], 0.0)
    o_ref[...] = jnp.dot(h.astype(jnp.bfloat16), wn_ref[...],
                         preferred_element_type=jnp.float32).astype(jnp.bfloat16)


def _layer_body(adj_ref, t_ref, b_ref, wn_ref, o_ref):
    acc = jnp.dot(adj_ref[...], t_ref[...], preferred_element_type=jnp.float32)
    h = jnp.maximum(acc + b_ref[0, :], 0.0)
    o_ref[...] = jnp.dot(h.astype(jnp.bfloat16), wn_ref[...],
                         preferred_element_type=jnp.float32).astype(jnp.bfloat16)


def _final_body(adj_ref, t_ref, b_ref, wn_ref, bd_ref, o_ref):
    acc = jnp.dot(adj_ref[...], t_ref[...], preferred_element_type=jnp.float32)
    h = jnp.maximum(acc + b_ref[0, :], 0.0)
    o_ref[...] = jnp.dot(h.astype(jnp.bfloat16), wn_ref[...],
                         preferred_element_type=jnp.float32) + bd_ref[0, :]


def _layer1(adj, t, b, w_next):
    n, f = t.shape
    fo = w_next.shape[1]
    return pl.pallas_call(
        _layer1_body,
        grid=(n // BM1,),
        in_specs=[
            pl.BlockSpec((BM1, n), lambda m: (m, 0)),
            pl.BlockSpec((n, f), lambda m: (0, 0)),
            pl.BlockSpec((1, f), lambda m: (0, 0)),
            pl.BlockSpec((f, fo), lambda m: (0, 0)),
        ],
        out_specs=[
            pl.BlockSpec((BM1, fo), lambda m: (m, 0)),
            pl.BlockSpec((BM1, n), lambda m: (m, 0)),
        ],
        out_shape=[
            jax.ShapeDtypeStruct((n, fo), jnp.bfloat16),
            jax.ShapeDtypeStruct((n, n), jnp.bfloat16),
        ],
        compiler_params=pltpu.CompilerParams(
            dimension_semantics=("arbitrary",)),
    )(adj, t, b, w_next)


def _layer(adj, t, b, w_next):
    n, f = t.shape
    fo = w_next.shape[1]
    return pl.pallas_call(
        _layer_body,
        grid=(n // BM2,),
        in_specs=[
            pl.BlockSpec((BM2, n), lambda m: (m, 0)),
            pl.BlockSpec((n, f), lambda m: (0, 0)),
            pl.BlockSpec((1, f), lambda m: (0, 0)),
            pl.BlockSpec((f, fo), lambda m: (0, 0)),
        ],
        out_specs=pl.BlockSpec((BM2, fo), lambda m: (m, 0)),
        out_shape=jax.ShapeDtypeStruct((n, fo), jnp.bfloat16),
        compiler_params=pltpu.CompilerParams(
            dimension_semantics=("arbitrary",)),
    )(adj, t, b, w_next)


def _final_layer(adj, t, b, wd, bd):
    n, f = t.shape
    fo = wd.shape[1]
    return pl.pallas_call(
        _final_body,
        grid=(n // BM2,),
        in_specs=[
            pl.BlockSpec((BM2, n), lambda m: (m, 0)),
            pl.BlockSpec((n, f), lambda m: (0, 0)),
            pl.BlockSpec((1, f), lambda m: (0, 0)),
            pl.BlockSpec((f, fo), lambda m: (0, 0)),
            pl.BlockSpec((1, fo), lambda m: (0, 0)),
        ],
        out_specs=pl.BlockSpec((BM2, fo), lambda m: (m, 0)),
        out_shape=jax.ShapeDtypeStruct((n, fo), jnp.float32),
        compiler_params=pltpu.CompilerParams(
            dimension_semantics=("arbitrary",)),
    )(adj, t, b, wd, bd)


def _xw(x, w):
    n, f = x.shape
    fo = w.shape[1]
    return pl.pallas_call(
        _xw_body,
        grid=(n // BM2,),
        in_specs=[
            pl.BlockSpec((BM2, f), lambda m: (m, 0)),
            pl.BlockSpec((f, fo), lambda m: (0, 0)),
        ],
        out_specs=pl.BlockSpec((BM2, fo), lambda m: (m, 0)),
        out_shape=jax.ShapeDtypeStruct((n, fo), jnp.bfloat16),
        compiler_params=pltpu.CompilerParams(
            dimension_semantics=("arbitrary",)),
    )(x, w)


def kernel(x, adj1, adj2, adj3, adj4, adj5, adj6, W1, b1, W2, b2, W3, b3,
           Wd, bd):
    w1b, w2b, w3b, wdb = (w.astype(jnp.bfloat16) for w in (W1, W2, W3, Wd))
    b1r = b1.reshape(1, -1)
    b2r = b2.reshape(1, -1)
    b3r = b3.reshape(1, -1)
    bdr = bd.reshape(1, -1)

    t1 = _xw(x, w1b)
    t2, adj_b = _layer1(adj5, t1, b1r, w2b)
    t3 = _layer(adj_b, t2, b2r, w3b)
    return _final_layer(adj_b, t3, b3r, wdb, bdr)
